# Initial kernel scaffold; baseline (speedup 1.0000x reference)
import jax
import jax.numpy as jnp
from jax.experimental import pallas as pl


def kernel(x, edge_index, edge_weight, W1, b1, W2, b2):
    raise NotImplementedError("write your pallas kernel here")


# speedup vs baseline: 15.8726x; 15.8726x over previous
import functools

import jax
import jax.numpy as jnp
from jax import lax
from jax.experimental import pallas as pl
from jax.experimental.pallas import tpu as pltpu
from jax.experimental.pallas import tpu_sc as plsc

N = 10000
NP = 10240
EP = 327680
CH = 128
NCH = (EP // 32) // CH
NC, NS, L = 2, 16, 16
RPT = NP // NS


def _mesh():
    return plsc.VectorSubcoreMesh(
        core_axis_name="c", subcore_axis_name="s",
        num_cores=NC, num_subcores=NS)


_SC_PARAMS = pltpu.CompilerParams(needs_layout_passes=False)
_SC_PARAMS_UNTILED = pltpu.CompilerParams(
    needs_layout_passes=False, use_tc_tiling_on_sc=False)


def _rsqrt16(x):
    i = lax.bitcast_convert_type(x, jnp.int32)
    i = jnp.full((L,), 0x5F3759DF, jnp.int32) - lax.shift_right_logical(i, 1)
    y = lax.bitcast_convert_type(i, jnp.float32)
    for _ in range(3):
        y = y * (1.5 - 0.5 * x * y * y)
    return y


def _deg_call(colr, ewr):
    @functools.partial(
        pl.kernel,
        out_type=jax.ShapeDtypeStruct((NC, NP), jnp.float32),
        mesh=_mesh(),
        compiler_params=_SC_PARAMS,
        scratch_types=[
            pltpu.VMEM((NCH, CH), jnp.int32),
            pltpu.VMEM((NCH, CH), jnp.float32),
            pltpu.VMEM((RPT,), jnp.float32),
            pltpu.VMEM_SHARED((NP,), jnp.float32),
        ],
    )
    def deg_kernel(col_hbm, ew_hbm, out_hbm, col_v, ew_v, zb_v, acc_sh):
        cid = lax.axis_index("c")
        sid = lax.axis_index("s")
        wid = cid * NS + sid
        pltpu.sync_copy(col_hbm.at[pl.ds(wid * NCH, NCH)], col_v)
        pltpu.sync_copy(ew_hbm.at[pl.ds(wid * NCH, NCH)], ew_v)

        def zb(k, carry):
            zb_v[pl.ds(k * L, L)] = jnp.zeros((L,), jnp.float32)
            return carry
        lax.fori_loop(0, RPT // L, zb, 0)
        pltpu.sync_copy(zb_v, acc_sh.at[pl.ds(sid * RPT, RPT)])
        plsc.subcore_barrier()

        def chunk(ch, carry):
            pltpu.sync_copy(ew_v.at[ch], acc_sh.at[col_v.at[ch]], add=True)
            return carry
        lax.fori_loop(0, NCH, chunk, 0)
        plsc.subcore_barrier()
        pltpu.sync_copy(acc_sh.at[pl.ds(sid * RPT, RPT)],
                        out_hbm.at[cid, pl.ds(sid * RPT, RPT)])

    return deg_kernel(colr, ewr)


def _norm_call(rowr, colr, ewr, deg):
    @functools.partial(
        pl.kernel,
        out_type=jax.ShapeDtypeStruct((EP // CH, CH), jnp.float32),
        mesh=_mesh(),
        compiler_params=_SC_PARAMS,
        scratch_types=[
            pltpu.VMEM((NCH, CH), jnp.int32),
            pltpu.VMEM((NCH, CH), jnp.int32),
            pltpu.VMEM((NCH, CH), jnp.float32),
            pltpu.VMEM((NC, NP), jnp.float32),
            pltpu.VMEM((NP,), jnp.float32),
        ],
    )
    def norm_kernel(row_hbm, col_hbm, ew_hbm, deg_hbm, s_hbm,
                    row_v, col_v, ew_v, deg_v, dinv_v):
        cid = lax.axis_index("c")
        sid = lax.axis_index("s")
        wid = cid * NS + sid
        pltpu.sync_copy(row_hbm.at[pl.ds(wid * NCH, NCH)], row_v)
        pltpu.sync_copy(col_hbm.at[pl.ds(wid * NCH, NCH)], col_v)
        pltpu.sync_copy(ew_hbm.at[pl.ds(wid * NCH, NCH)], ew_v)
        pltpu.sync_copy(deg_hbm, deg_v)

        def dbody(k, carry):
            sl = pl.ds(k * L, L)
            d = deg_v[0, sl] + deg_v[1, sl] + 1.0
            dinv_v[sl] = _rsqrt16(d)
            return carry
        lax.fori_loop(0, NP // L, dbody, 0)

        def chunk(ch, carry):
            for sub in range(CH // L):
                sl = pl.ds(sub * L, L)
                rr = row_v[ch, sl]
                cc = col_v[ch, sl]
                w = ew_v[ch, sl]
                ew_v[ch, sl] = (plsc.load_gather(dinv_v, [rr]) * w *
                                plsc.load_gather(dinv_v, [cc]))
            return carry
        lax.fori_loop(0, NCH, chunk, 0)
        pltpu.sync_copy(ew_v, s_hbm.at[pl.ds(wid * NCH, NCH)])

    return norm_kernel(rowr, colr, ewr, deg)


GS = 16


def _agg_call(F, g, rowr, colr, sr):
    @functools.partial(
        pl.kernel,
        out_type=jax.ShapeDtypeStruct((NC, NP, F), jnp.float32),
        mesh=_mesh(),
        compiler_params=_SC_PARAMS if F % 128 == 0 else _SC_PARAMS_UNTILED,
        scratch_types=[
            pltpu.VMEM((GS, CH), jnp.int32),
            pltpu.VMEM((GS, CH), jnp.int32),
            pltpu.VMEM((GS, CH), jnp.float32),
            pltpu.VMEM((CH, F), jnp.float32),
            pltpu.VMEM_SHARED((NP, F), jnp.float32),
            pltpu.SemaphoreType.DMA,
        ],
    )
    def agg_kernel(g_hbm, row_hbm, col_hbm, s_hbm, out_hbm,
                   row_v, col_v, s_v, msg_v, acc_sh, sem):
        cid = lax.axis_index("c")
        sid = lax.axis_index("s")
        wid = cid * NS + sid

        def zrow(r, carry):
            for gg in range(F // L):
                msg_v[r, pl.ds(gg * L, L)] = jnp.zeros((L,), jnp.float32)
            return carry
        lax.fori_loop(0, CH, zrow, 0)
        for k in range(RPT // CH):
            pltpu.sync_copy(msg_v, acc_sh.at[pl.ds(sid * RPT + k * CH, CH)])
        plsc.subcore_barrier()

        def group(grp, carry):
            base = wid * NCH + grp * GS
            pltpu.sync_copy(row_hbm.at[pl.ds(base, GS)], row_v)
            pltpu.sync_copy(col_hbm.at[pl.ds(base, GS)], col_v)
            pltpu.sync_copy(s_hbm.at[pl.ds(base, GS)], s_v)

            def chunk(j, c1):
                pltpu.async_copy(g_hbm.at[row_v.at[j]], msg_v, sem).wait()

                def rbody(r, c2):
                    sb = plsc.load_gather(
                        s_v, [jnp.full((L,), j, jnp.int32),
                              jnp.full((L,), r, jnp.int32)])
                    for gg in range(F // L):
                        sl = pl.ds(gg * L, L)
                        msg_v[r, sl] = msg_v[r, sl] * sb
                    return c2
                lax.fori_loop(0, CH, rbody, 0)
                pltpu.sync_copy(msg_v, acc_sh.at[col_v.at[j]], add=True)
                return c1
            lax.fori_loop(0, GS, chunk, 0)
            return carry
        lax.fori_loop(0, NCH // GS, group, 0)
        plsc.subcore_barrier()
        pltpu.sync_copy(acc_sh.at[pl.ds(sid * RPT, RPT)],
                        out_hbm.at[cid, pl.ds(sid * RPT, RPT)])

    return agg_kernel(g, rowr, colr, sr)


def _mm_call(x, w):
    def body(x_ref, w_ref, o_ref):
        o_ref[...] = jnp.dot(x_ref[...], w_ref[...],
                             preferred_element_type=jnp.float32)
    return pl.pallas_call(
        body,
        out_shape=jax.ShapeDtypeStruct((x.shape[0], w.shape[1]), jnp.float32),
    )(x, w)


def _mid_call(degT, a0, a1, g1, b1r, W2p):
    def body(d_ref, a0_ref, a1_ref, g_ref, b_ref, w_ref, o_ref):
        inv = 1.0 / (d_ref[:, 0:1] + d_ref[:, 1:2] + 1.0)
        z = a0_ref[...] + a1_ref[...] + g_ref[...] * inv + b_ref[...]
        h = jnp.maximum(z, 0.0)
        o_ref[...] = jnp.dot(h, w_ref[...],
                             preferred_element_type=jnp.float32)
    return pl.pallas_call(
        body,
        out_shape=jax.ShapeDtypeStruct((NP, W2p.shape[1]), jnp.float32),
    )(degT, a0, a1, g1, b1r, W2p)


def _final_call(degT, a0, a1, g2, b2r):
    F2 = b2r.shape[1]
    def body(d_ref, a0_ref, a1_ref, g_ref, b_ref, o_ref):
        inv = 1.0 / (d_ref[:, 0:1] + d_ref[:, 1:2] + 1.0)
        z = (a0_ref[...] + a1_ref[...] + g_ref[...] * inv)[:, :F2] + b_ref[...]
        m = jnp.max(z, axis=1, keepdims=True)
        e = jnp.exp(z - m)
        s = jnp.sum(e, axis=1, keepdims=True)
        o_ref[...] = z - m - jnp.log(s)
    return pl.pallas_call(
        body,
        out_shape=jax.ShapeDtypeStruct((NP, F2), jnp.float32),
    )(degT, a0, a1, g2, b2r)


def kernel(x, edge_index, edge_weight, W1, b1, W2, b2):
    row = edge_index[0].astype(jnp.int32)
    col = edge_index[1].astype(jnp.int32)
    ew = edge_weight.astype(jnp.float32)
    pad = EP - row.shape[0]
    pidx = (jnp.arange(pad, dtype=jnp.int32) * 37) % N
    rowp = jnp.concatenate([row, pidx]).reshape(EP // CH, CH)
    colp = jnp.concatenate([col, pidx]).reshape(EP // CH, CH)
    ewp = jnp.concatenate([ew, jnp.zeros((pad,), jnp.float32)]
                          ).reshape(EP // CH, CH)
    xp = jnp.concatenate(
        [x, jnp.zeros((NP - N, x.shape[1]), jnp.float32)], axis=0)
    F2P = 64
    W2p = jnp.concatenate(
        [W2, jnp.zeros((W2.shape[0], F2P - W2.shape[1]), jnp.float32)], axis=1)

    deg = _deg_call(colp, ewp)
    degT = deg.T
    sp = _norm_call(rowp, colp, ewp, deg)
    g1 = _mm_call(xp, W1)
    agg1 = _agg_call(128, g1, rowp, colp, sp)
    g2 = _mid_call(degT, agg1[0], agg1[1], g1,
                   b1.reshape(1, -1), W2p)
    agg2 = _agg_call(F2P, g2, rowp, colp, sp)
    out = _final_call(degT, agg2[0], agg2[1], g2, b2.reshape(1, -1))
    return out[:N]
